# Initial kernel scaffold; baseline (speedup 1.0000x reference)
import jax
import jax.numpy as jnp
from jax.experimental import pallas as pl


def kernel(x, edge_index, W1_l, W1_r, b1, W2_l, W2_r, b2):
    raise NotImplementedError("write your pallas kernel here")


# speedup vs baseline: 5.2417x; 5.2417x over previous
import functools

import jax
import jax.numpy as jnp
from jax import lax
from jax.experimental import pallas as pl
from jax.experimental.pallas import tpu as pltpu
from jax.experimental.pallas import tpu_sc as plsc

N = 10000
E = 320000
D = 128
H = 128
OUT = 2

NC = 2
NS = 16
K = 80
EPW = E // (NC * NS)
NCHUNK = EPW // K
NP = 10240
RPT = NP // NS
RCH = 64
NRC = RPT // RCH


def _make_segsum(width, tc_tiling):
    mesh = plsc.VectorSubcoreMesh(core_axis_name="c", subcore_axis_name="s")
    out_type = jax.ShapeDtypeStruct((NC, NP, width), jnp.float32)
    scratch = [
        pltpu.VMEM((K,), jnp.int32),
        pltpu.VMEM((K,), jnp.int32),
        pltpu.VMEM((K, width), jnp.float32),
        pltpu.VMEM((RCH, width), jnp.float32),
        pltpu.VMEM_SHARED((NP, width), jnp.float32),
        pltpu.SemaphoreType.DMA,
    ]

    def body(x_hbm, src_hbm, dst_hbm, s_out, src_i, dst_i, rows, stage,
             acc, sem):
        c = lax.axis_index("c")
        s = lax.axis_index("s")

        zv = jnp.zeros((16,), jnp.float32)

        def zrow(i, _):
            for g in range(width // 16):
                stage[i, pl.ds(g * 16, 16)] = zv
            return 0
        lax.fori_loop(0, RCH, zrow, 0)

        for r in range(NRC):
            off = s * RPT + r * RCH
            pltpu.sync_copy(stage, acc.at[pl.ds(off, RCH)])
        plsc.subcore_barrier()

        wbase = (c * NS + s) * EPW

        def chunk(j, _):
            base = pl.multiple_of(wbase + j * K, 8)
            pltpu.sync_copy(src_hbm.at[pl.ds(base, K)], src_i)
            pltpu.sync_copy(dst_hbm.at[pl.ds(base, K)], dst_i)
            pltpu.async_copy(x_hbm.at[src_i], rows, sem).wait()
            pltpu.sync_copy(rows, acc.at[dst_i], add=True)
            return 0
        lax.fori_loop(0, NCHUNK, chunk, 0)
        plsc.subcore_barrier()

        for r in range(NRC):
            off = s * RPT + r * RCH
            pltpu.sync_copy(acc.at[pl.ds(off, RCH)], stage)
            pltpu.sync_copy(stage, s_out.at[c, pl.ds(off, RCH)])

    params = pltpu.CompilerParams(use_tc_tiling_on_sc=tc_tiling)
    return pl.kernel(body, out_type=out_type, mesh=mesh,
                     scratch_types=scratch, compiler_params=params)


def _make_counts():
    mesh = plsc.VectorSubcoreMesh(core_axis_name="c", subcore_axis_name="s")
    out_type = jax.ShapeDtypeStruct((NC, NP, 16), jnp.float32)
    scratch = [
        pltpu.VMEM((K,), jnp.int32),
        pltpu.VMEM((K, 16), jnp.float32),
        pltpu.VMEM((RCH, 16), jnp.float32),
        pltpu.VMEM_SHARED((NP, 16), jnp.float32),
    ]

    def body(dst_hbm, c_out, dst_i, ones, stage, cacc):
        c = lax.axis_index("c")
        s = lax.axis_index("s")
        zv = jnp.zeros((16,), jnp.float32)
        ov = jnp.ones((16,), jnp.float32)

        def zrow(i, _):
            stage[i, :] = zv
            return 0
        lax.fori_loop(0, RCH, zrow, 0)

        def orow(i, _):
            ones[i, :] = ov
            return 0
        lax.fori_loop(0, K, orow, 0)

        for r in range(NRC):
            off = s * RPT + r * RCH
            pltpu.sync_copy(stage, cacc.at[pl.ds(off, RCH)])
        plsc.subcore_barrier()

        wbase = (c * NS + s) * EPW

        def chunk(j, _):
            base = pl.multiple_of(wbase + j * K, 8)
            pltpu.sync_copy(dst_hbm.at[pl.ds(base, K)], dst_i)
            pltpu.sync_copy(ones, cacc.at[dst_i], add=True)
            return 0
        lax.fori_loop(0, NCHUNK, chunk, 0)
        plsc.subcore_barrier()

        for r in range(NRC):
            off = s * RPT + r * RCH
            pltpu.sync_copy(cacc.at[pl.ds(off, RCH)], stage)
            pltpu.sync_copy(stage, c_out.at[c, pl.ds(off, RCH)])

    params = pltpu.CompilerParams(use_tc_tiling_on_sc=False)
    return pl.kernel(body, out_type=out_type, mesh=mesh,
                     scratch_types=scratch, compiler_params=params)


_segsum128 = _make_segsum(D, True)
_segsum16 = _make_segsum(16, False)
_counts = _make_counts()

BN = 1000


def _tc1_body(s_ref, c_ref, x_ref, w1l_ref, w1r_ref, b1_ref, w2_ref, p_ref):
    S = s_ref[0] + s_ref[1]
    cnt = c_ref[0, :, 0:1] + c_ref[1, :, 0:1]
    agg = S / jnp.maximum(cnt, 1.0)
    t = jnp.dot(agg, w1l_ref[...], preferred_element_type=jnp.float32)
    t = t + jnp.dot(x_ref[...], w1r_ref[...], preferred_element_type=jnp.float32)
    t = t + b1_ref[...]
    nrm = jnp.sqrt(jnp.sum(t * t, axis=1, keepdims=True))
    h = jnp.maximum(t / jnp.maximum(nrm, 1e-12), 0.0)
    p_ref[...] = jnp.dot(h, w2_ref[...], preferred_element_type=jnp.float32)


def _tc1(S1p, C1p, x, W1_l, W1_r, b1r, W2cat):
    return pl.pallas_call(
        _tc1_body,
        grid=(N // BN,),
        in_specs=[
            pl.BlockSpec((NC, BN, D), lambda i: (0, i, 0)),
            pl.BlockSpec((NC, BN, 16), lambda i: (0, i, 0)),
            pl.BlockSpec((BN, D), lambda i: (i, 0)),
            pl.BlockSpec((D, H), lambda i: (0, 0)),
            pl.BlockSpec((D, H), lambda i: (0, 0)),
            pl.BlockSpec((1, H), lambda i: (0, 0)),
            pl.BlockSpec((H, 16), lambda i: (0, 0)),
        ],
        out_specs=pl.BlockSpec((BN, 16), lambda i: (i, 0)),
        out_shape=jax.ShapeDtypeStruct((N, 16), jnp.float32),
    )(S1p, C1p, x, W1_l, W1_r, b1r, W2cat)


def _tc2_body(s2_ref, c_ref, p_ref, b2_ref, o_ref):
    S2 = s2_ref[0] + s2_ref[1]
    cnt = c_ref[0, :, 0:1] + c_ref[1, :, 0:1]
    agg = S2[:, 0:2] / jnp.maximum(cnt, 1.0)
    v = agg + p_ref[:, 2:4] + b2_ref[...]
    nrm = jnp.sqrt(jnp.sum(v * v, axis=1, keepdims=True))
    v = v / jnp.maximum(nrm, 1e-12)
    m = jnp.max(v, axis=1, keepdims=True)
    e = jnp.exp(v - m)
    o_ref[...] = (v - m) - jnp.log(jnp.sum(e, axis=1, keepdims=True))


def _tc2(S2p, C1p, p2s, b2r):
    return pl.pallas_call(
        _tc2_body,
        grid=(N // BN,),
        in_specs=[
            pl.BlockSpec((NC, BN, 16), lambda i: (0, i, 0)),
            pl.BlockSpec((NC, BN, 16), lambda i: (0, i, 0)),
            pl.BlockSpec((BN, 16), lambda i: (i, 0)),
            pl.BlockSpec((1, OUT), lambda i: (0, 0)),
        ],
        out_specs=pl.BlockSpec((BN, OUT), lambda i: (i, 0)),
        out_shape=jax.ShapeDtypeStruct((N, OUT), jnp.float32),
    )(S2p, C1p, p2s, b2r)


def kernel(x, edge_index, W1_l, W1_r, b1, W2_l, W2_r, b2):
    src = edge_index[0]
    dst = edge_index[1]
    S1p = _segsum128(x, src, dst)
    C1p = _counts(dst)
    W2cat = (jnp.zeros((H, 16), jnp.float32)
             .at[:, 0:2].set(W2_l).at[:, 2:4].set(W2_r))
    p2s = _tc1(S1p, C1p, x, W1_l, W1_r, b1.reshape(1, H), W2cat)
    S2p = _segsum16(p2s, src, dst)
    if isinstance(S2p, (list, tuple)):
        S2p = S2p[0]
    out = _tc2(S2p, C1p, p2s, b2.reshape(1, OUT))
    return out


# speedup vs baseline: 13.8271x; 2.6379x over previous
import functools

import jax
import jax.numpy as jnp
from jax import lax
from jax.experimental import pallas as pl
from jax.experimental.pallas import tpu as pltpu
from jax.experimental.pallas import tpu_sc as plsc

N = 10000
E = 320000
D = 128
H = 128
OUT = 2

NC = 2
NS = 16
NW = NC * NS
CH = 128
ER = E // CH
CPW = 78
SLAB = 26
NSLAB = CPW // SLAB
TAILR = NW * CPW
NP = 10240
RPT = NP // NS
ZCH = 16
OCH = 128


def _zero_acc(stage, acc, s, width, sem):
    zv = jnp.zeros((16,), jnp.float32)

    def zrow(i, _):
        for g in range(width // 16):
            stage[i, pl.ds(g * 16, 16)] = zv
        return 0
    lax.fori_loop(0, ZCH, zrow, 0)
    hs = []
    for r in range(RPT // ZCH):
        hs.append(pltpu.async_copy(
            stage, acc.at[pl.ds(s * RPT + r * ZCH, ZCH)], sem))
    for h in hs:
        h.wait()


def _copy_out(acc, out, c, s, sem):
    hs = []
    for r in range(RPT // OCH):
        off = s * RPT + r * OCH
        hs.append(pltpu.async_copy(acc.at[pl.ds(off, OCH)],
                                   out.at[c, pl.ds(off, OCH)], sem))
    for h in hs:
        h.wait()


def _make_segsum(width):
    mesh = plsc.VectorSubcoreMesh(core_axis_name="c", subcore_axis_name="s")
    out_type = jax.ShapeDtypeStruct((NC, NP, width), jnp.float32)
    scratch = [
        pltpu.VMEM((SLAB, CH), jnp.int32),
        pltpu.VMEM((SLAB, CH), jnp.int32),
        pltpu.VMEM((CH, width), jnp.float32),
        pltpu.VMEM((CH, width), jnp.float32),
        pltpu.VMEM((ZCH, width), jnp.float32),
        pltpu.VMEM_SHARED((NP, width), jnp.float32),
        pltpu.SemaphoreType.DMA,
        pltpu.SemaphoreType.DMA,
        pltpu.SemaphoreType.DMA,
        pltpu.SemaphoreType.DMA,
        pltpu.SemaphoreType.DMA,
    ]

    def body(x_hbm, src_hbm, dst_hbm, s_out, srcs, dsts, rows0, rows1,
             stage, acc, sg0, sg1, ss0, ss1, sz):
        c = lax.axis_index("c")
        s = lax.axis_index("s")
        w = c * NS + s
        rows = (rows0, rows1)
        sg = (sg0, sg1)
        ss = (ss0, ss1)

        _zero_acc(stage, acc, s, width, sz)
        plsc.subcore_barrier()

        wrow = w * CPW

        def slab(t, _):
            rbase = wrow + t * SLAB
            pltpu.sync_copy(src_hbm.at[pl.ds(rbase, SLAB)], srcs)
            pltpu.sync_copy(dst_hbm.at[pl.ds(rbase, SLAB)], dsts)
            gd = [None, None]
            sd = [None, None]
            gd[0] = pltpu.async_copy(x_hbm.at[srcs.at[0]], rows0, sg0)
            for b in range(SLAB):
                cur = b % 2
                nxt = 1 - cur
                if b + 1 < SLAB:
                    if sd[nxt] is not None:
                        sd[nxt].wait()
                    gd[nxt] = pltpu.async_copy(
                        x_hbm.at[srcs.at[b + 1]], rows[nxt], sg[nxt])
                gd[cur].wait()
                sd[cur] = pltpu.async_copy(
                    rows[cur], acc.at[dsts.at[b]], ss[cur], add=True)
            for slot in range(2):
                if sd[slot] is not None:
                    sd[slot].wait()
            return 0
        lax.fori_loop(0, NSLAB, slab, 0)

        @pl.when(w < 4)
        def _():
            pltpu.sync_copy(src_hbm.at[pl.ds(TAILR + w, 1)], srcs.at[pl.ds(0, 1)])
            pltpu.sync_copy(dst_hbm.at[pl.ds(TAILR + w, 1)], dsts.at[pl.ds(0, 1)])
            pltpu.async_copy(x_hbm.at[srcs.at[0]], rows0, sg0).wait()
            pltpu.sync_copy(rows0, acc.at[dsts.at[0]], add=True)

        plsc.subcore_barrier()
        _copy_out(acc, s_out, c, s, sz)

    params = pltpu.CompilerParams(use_tc_tiling_on_sc=False)
    return pl.kernel(body, out_type=out_type, mesh=mesh,
                     scratch_types=scratch, compiler_params=params)


def _make_counts():
    mesh = plsc.VectorSubcoreMesh(core_axis_name="c", subcore_axis_name="s")
    out_type = jax.ShapeDtypeStruct((NC, NP, 16), jnp.float32)
    scratch = [
        pltpu.VMEM((SLAB, CH), jnp.int32),
        pltpu.VMEM((CH, 16), jnp.float32),
        pltpu.VMEM((ZCH, 16), jnp.float32),
        pltpu.VMEM_SHARED((NP, 16), jnp.float32),
        pltpu.SemaphoreType.DMA,
        pltpu.SemaphoreType.DMA,
    ]

    def body(dst_hbm, c_out, dsts, ones, stage, cacc, sc, sz):
        c = lax.axis_index("c")
        s = lax.axis_index("s")
        w = c * NS + s
        ov = jnp.ones((16,), jnp.float32)

        def orow(i, _):
            ones[i, :] = ov
            return 0
        lax.fori_loop(0, CH, orow, 0)
        _zero_acc(stage, cacc, s, 16, sz)
        plsc.subcore_barrier()

        wrow = w * CPW

        def slab(t, _):
            rbase = wrow + t * SLAB
            pltpu.sync_copy(dst_hbm.at[pl.ds(rbase, SLAB)], dsts)
            sd = []
            for b in range(SLAB):
                sd.append(pltpu.async_copy(ones, cacc.at[dsts.at[b]], sc,
                                           add=True))
            for h in sd:
                h.wait()
            return 0
        lax.fori_loop(0, NSLAB, slab, 0)

        @pl.when(w < 4)
        def _():
            pltpu.sync_copy(dst_hbm.at[pl.ds(TAILR + w, 1)], dsts.at[pl.ds(0, 1)])
            pltpu.sync_copy(ones, cacc.at[dsts.at[0]], add=True)

        plsc.subcore_barrier()
        _copy_out(cacc, c_out, c, s, sz)

    params = pltpu.CompilerParams(use_tc_tiling_on_sc=False)
    return pl.kernel(body, out_type=out_type, mesh=mesh,
                     scratch_types=scratch, compiler_params=params)


_segsum128 = _make_segsum(D)
_segsum16 = _make_segsum(16)
_counts = _make_counts()

BN = 1000


def _tc1_body(s_ref, c_ref, x_ref, w1l_ref, w1r_ref, b1_ref, w2_ref, p_ref):
    S = s_ref[0] + s_ref[1]
    cnt = c_ref[0, :, 0:1] + c_ref[1, :, 0:1]
    agg = S / jnp.maximum(cnt, 1.0)
    t = jnp.dot(agg, w1l_ref[...], preferred_element_type=jnp.float32)
    t = t + jnp.dot(x_ref[...], w1r_ref[...], preferred_element_type=jnp.float32)
    t = t + b1_ref[...]
    nrm = jnp.sqrt(jnp.sum(t * t, axis=1, keepdims=True))
    h = jnp.maximum(t / jnp.maximum(nrm, 1e-12), 0.0)
    p_ref[...] = jnp.dot(h, w2_ref[...], preferred_element_type=jnp.float32)


def _tc1(S1p, C1p, x, W1_l, W1_r, b1r, W2cat):
    return pl.pallas_call(
        _tc1_body,
        grid=(N // BN,),
        in_specs=[
            pl.BlockSpec((NC, BN, D), lambda i: (0, i, 0)),
            pl.BlockSpec((NC, BN, 16), lambda i: (0, i, 0)),
            pl.BlockSpec((BN, D), lambda i: (i, 0)),
            pl.BlockSpec((D, H), lambda i: (0, 0)),
            pl.BlockSpec((D, H), lambda i: (0, 0)),
            pl.BlockSpec((1, H), lambda i: (0, 0)),
            pl.BlockSpec((H, 16), lambda i: (0, 0)),
        ],
        out_specs=pl.BlockSpec((BN, 16), lambda i: (i, 0)),
        out_shape=jax.ShapeDtypeStruct((N, 16), jnp.float32),
    )(S1p, C1p, x, W1_l, W1_r, b1r, W2cat)


def _tc2_body(s2_ref, c_ref, p_ref, b2_ref, o_ref):
    S2 = s2_ref[0] + s2_ref[1]
    cnt = c_ref[0, :, 0:1] + c_ref[1, :, 0:1]
    agg = S2[:, 0:2] / jnp.maximum(cnt, 1.0)
    v = agg + p_ref[:, 2:4] + b2_ref[...]
    nrm = jnp.sqrt(jnp.sum(v * v, axis=1, keepdims=True))
    v = v / jnp.maximum(nrm, 1e-12)
    m = jnp.max(v, axis=1, keepdims=True)
    e = jnp.exp(v - m)
    o_ref[...] = (v - m) - jnp.log(jnp.sum(e, axis=1, keepdims=True))


def _tc2(S2p, C1p, p2s, b2r):
    return pl.pallas_call(
        _tc2_body,
        grid=(N // BN,),
        in_specs=[
            pl.BlockSpec((NC, BN, 16), lambda i: (0, i, 0)),
            pl.BlockSpec((NC, BN, 16), lambda i: (0, i, 0)),
            pl.BlockSpec((BN, 16), lambda i: (i, 0)),
            pl.BlockSpec((1, OUT), lambda i: (0, 0)),
        ],
        out_specs=pl.BlockSpec((BN, OUT), lambda i: (i, 0)),
        out_shape=jax.ShapeDtypeStruct((N, OUT), jnp.float32),
    )(S2p, C1p, p2s, b2r)


def kernel(x, edge_index, W1_l, W1_r, b1, W2_l, W2_r, b2):
    src = edge_index[0].reshape(ER, CH)
    dst = edge_index[1].reshape(ER, CH)
    S1p = _segsum128(x, src, dst)
    C1p = _counts(dst)
    W2cat = (jnp.zeros((H, 16), jnp.float32)
             .at[:, 0:2].set(W2_l).at[:, 2:4].set(W2_r))
    p2s = _tc1(S1p, C1p, x, W1_l, W1_r, b1.reshape(1, H), W2cat)
    S2p = _segsum16(p2s, src, dst)
    if isinstance(S2p, (list, tuple)):
        S2p = S2p[0]
    out = _tc2(S2p, C1p, p2s, b2.reshape(1, OUT))
    return out


# speedup vs baseline: 15.1346x; 1.0946x over previous
import functools

import jax
import jax.numpy as jnp
from jax import lax
from jax.experimental import pallas as pl
from jax.experimental.pallas import tpu as pltpu
from jax.experimental.pallas import tpu_sc as plsc

N = 10000
E = 320000
D = 128
H = 128
OUT = 2

NC = 2
NS = 16
NW = NC * NS
CH = 128
ER = E // CH
CPW = 78
SLAB = 26
NSLAB = CPW // SLAB
TAILR = NW * CPW
NP = 10240
RPT = NP // NS
ZCH = 16
OCH = 128


def _zero_acc(stage, acc, s, width, sem):
    zv = jnp.zeros((16,), jnp.float32)

    def zrow(i, _):
        for g in range(width // 16):
            stage[i, pl.ds(g * 16, 16)] = zv
        return 0
    lax.fori_loop(0, ZCH, zrow, 0)
    hs = []
    for r in range(RPT // ZCH):
        hs.append(pltpu.async_copy(
            stage, acc.at[pl.ds(s * RPT + r * ZCH, ZCH)], sem))
    for h in hs:
        h.wait()


def _copy_out(acc, out, c, s, sem):
    hs = []
    for r in range(RPT // OCH):
        off = s * RPT + r * OCH
        hs.append(pltpu.async_copy(acc.at[pl.ds(off, OCH)],
                                   out.at[c, pl.ds(off, OCH)], sem))
    for h in hs:
        h.wait()


def _make_segsum(width):
    mesh = plsc.VectorSubcoreMesh(core_axis_name="c", subcore_axis_name="s")
    out_type = jax.ShapeDtypeStruct((NC, NP, width), jnp.float32)
    scratch = [
        pltpu.VMEM((SLAB, CH), jnp.int32),
        pltpu.VMEM((SLAB, CH), jnp.int32),
        pltpu.VMEM((CH, width), jnp.float32),
        pltpu.VMEM((CH, width), jnp.float32),
        pltpu.VMEM((ZCH, width), jnp.float32),
        pltpu.VMEM_SHARED((NP, width), jnp.float32),
        pltpu.SemaphoreType.DMA,
        pltpu.SemaphoreType.DMA,
        pltpu.SemaphoreType.DMA,
        pltpu.SemaphoreType.DMA,
        pltpu.SemaphoreType.DMA,
    ]

    def body(x_hbm, src_hbm, dst_hbm, s_out, srcs, dsts, rows0, rows1,
             stage, acc, sg0, sg1, ss0, ss1, sz):
        c = lax.axis_index("c")
        s = lax.axis_index("s")
        w = c * NS + s
        rows = (rows0, rows1)
        sg = (sg0, sg1)
        ss = (ss0, ss1)

        _zero_acc(stage, acc, s, width, sz)
        plsc.subcore_barrier()

        wrow = w * CPW

        def slab(t, _):
            rbase = wrow + t * SLAB
            pltpu.sync_copy(src_hbm.at[pl.ds(rbase, SLAB)], srcs)
            pltpu.sync_copy(dst_hbm.at[pl.ds(rbase, SLAB)], dsts)
            gd = [None, None]
            sd = [None, None]
            gd[0] = pltpu.async_copy(x_hbm.at[srcs.at[0]], rows0, sg0)
            for b in range(SLAB):
                cur = b % 2
                nxt = 1 - cur
                if b + 1 < SLAB:
                    if sd[nxt] is not None:
                        sd[nxt].wait()
                    gd[nxt] = pltpu.async_copy(
                        x_hbm.at[srcs.at[b + 1]], rows[nxt], sg[nxt])
                gd[cur].wait()
                sd[cur] = pltpu.async_copy(
                    rows[cur], acc.at[dsts.at[b]], ss[cur], add=True)
            for slot in range(2):
                if sd[slot] is not None:
                    sd[slot].wait()
            return 0
        lax.fori_loop(0, NSLAB, slab, 0)

        @pl.when(w < 4)
        def _():
            pltpu.sync_copy(src_hbm.at[pl.ds(TAILR + w, 1)], srcs.at[pl.ds(0, 1)])
            pltpu.sync_copy(dst_hbm.at[pl.ds(TAILR + w, 1)], dsts.at[pl.ds(0, 1)])
            pltpu.async_copy(x_hbm.at[srcs.at[0]], rows0, sg0).wait()
            pltpu.sync_copy(rows0, acc.at[dsts.at[0]], add=True)

        plsc.subcore_barrier()
        _copy_out(acc, s_out, c, s, sz)

    params = pltpu.CompilerParams(use_tc_tiling_on_sc=False)
    return pl.kernel(body, out_type=out_type, mesh=mesh,
                     scratch_types=scratch, compiler_params=params)


def _make_counts():
    mesh = plsc.VectorSubcoreMesh(core_axis_name="c", subcore_axis_name="s")
    out_type = jax.ShapeDtypeStruct((NC, NP, 16), jnp.float32)
    scratch = [
        pltpu.VMEM((SLAB, CH), jnp.int32),
        pltpu.VMEM((CH, 16), jnp.float32),
        pltpu.VMEM((ZCH, 16), jnp.float32),
        pltpu.VMEM_SHARED((NP, 16), jnp.float32),
        pltpu.SemaphoreType.DMA,
        pltpu.SemaphoreType.DMA,
    ]

    def body(dst_hbm, c_out, dsts, ones, stage, cacc, sc, sz):
        c = lax.axis_index("c")
        s = lax.axis_index("s")
        w = c * NS + s
        ov = jnp.ones((16,), jnp.float32)

        def orow(i, _):
            ones[i, :] = ov
            return 0
        lax.fori_loop(0, CH, orow, 0)
        _zero_acc(stage, cacc, s, 16, sz)
        plsc.subcore_barrier()

        wrow = w * CPW

        def slab(t, _):
            rbase = wrow + t * SLAB
            pltpu.sync_copy(dst_hbm.at[pl.ds(rbase, SLAB)], dsts)
            sd = []
            for b in range(SLAB):
                sd.append(pltpu.async_copy(ones, cacc.at[dsts.at[b]], sc,
                                           add=True))
            for h in sd:
                h.wait()
            return 0
        lax.fori_loop(0, NSLAB, slab, 0)

        @pl.when(w < 4)
        def _():
            pltpu.sync_copy(dst_hbm.at[pl.ds(TAILR + w, 1)], dsts.at[pl.ds(0, 1)])
            pltpu.sync_copy(ones, cacc.at[dsts.at[0]], add=True)

        plsc.subcore_barrier()
        _copy_out(cacc, c_out, c, s, sz)

    params = pltpu.CompilerParams(use_tc_tiling_on_sc=False)
    return pl.kernel(body, out_type=out_type, mesh=mesh,
                     scratch_types=scratch, compiler_params=params)


def _make_segsum16():
    NSLOT = 4
    mesh = plsc.VectorSubcoreMesh(core_axis_name="c", subcore_axis_name="s")
    out_type = jax.ShapeDtypeStruct((NC, NP, 16), jnp.float32)
    scratch = [
        pltpu.VMEM((CPW, CH), jnp.int32),
        pltpu.VMEM((CPW, CH), jnp.int32),
        [pltpu.VMEM((CH, 16), jnp.float32) for _ in range(NSLOT)],
        pltpu.VMEM((ZCH, 16), jnp.float32),
        pltpu.VMEM_SHARED((N, 16), jnp.float32),
        pltpu.VMEM_SHARED((NP, 16), jnp.float32),
        [pltpu.SemaphoreType.DMA for _ in range(NSLOT)],
        [pltpu.SemaphoreType.DMA for _ in range(NSLOT)],
        pltpu.SemaphoreType.DMA,
    ]
    TPT = N // NS

    def body(x_hbm, src_hbm, dst_hbm, s_out, srcs, dsts, rows, stage,
             tbl, acc, sg, ss, sz):
        c = lax.axis_index("c")
        s = lax.axis_index("s")
        w = c * NS + s

        _zero_acc(stage, acc, s, 16, sz)
        pltpu.async_copy(x_hbm.at[pl.ds(s * TPT, TPT)],
                         tbl.at[pl.ds(s * TPT, TPT)], sz).wait()
        wrow = w * CPW
        pltpu.sync_copy(src_hbm.at[pl.ds(wrow, CPW)], srcs)
        pltpu.sync_copy(dst_hbm.at[pl.ds(wrow, CPW)], dsts)
        plsc.subcore_barrier()

        LA = NSLOT - 1
        gd = [None] * NSLOT
        sd = [None] * NSLOT
        for k in range(LA):
            gd[k] = pltpu.async_copy(tbl.at[srcs.at[k]], rows[k], sg[k])
        for b in range(CPW):
            cur = b % NSLOT
            nb = b + LA
            if nb < CPW:
                slot = nb % NSLOT
                if sd[slot] is not None:
                    sd[slot].wait()
                gd[slot] = pltpu.async_copy(
                    tbl.at[srcs.at[nb]], rows[slot], sg[slot])
            gd[cur].wait()
            sd[cur] = pltpu.async_copy(
                rows[cur], acc.at[dsts.at[b]], ss[cur], add=True)
        for slot in range(NSLOT):
            if sd[slot] is not None:
                sd[slot].wait()

        @pl.when(w < 4)
        def _():
            pltpu.sync_copy(src_hbm.at[pl.ds(TAILR + w, 1)], srcs.at[pl.ds(0, 1)])
            pltpu.sync_copy(dst_hbm.at[pl.ds(TAILR + w, 1)], dsts.at[pl.ds(0, 1)])
            pltpu.async_copy(tbl.at[srcs.at[0]], rows[0], sg[0]).wait()
            pltpu.sync_copy(rows[0], acc.at[dsts.at[0]], add=True)

        plsc.subcore_barrier()
        _copy_out(acc, s_out, c, s, sz)

    params = pltpu.CompilerParams(use_tc_tiling_on_sc=False)
    return pl.kernel(body, out_type=out_type, mesh=mesh,
                     scratch_types=scratch, compiler_params=params)


_segsum128 = _make_segsum(D)
_segsum16 = _make_segsum16()
_counts = _make_counts()

BN = 1000


def _tc1_body(s_ref, c_ref, x_ref, w1l_ref, w1r_ref, b1_ref, w2_ref, p_ref):
    S = s_ref[0] + s_ref[1]
    cnt = c_ref[0, :, 0:1] + c_ref[1, :, 0:1]
    agg = S / jnp.maximum(cnt, 1.0)
    t = jnp.dot(agg, w1l_ref[...], preferred_element_type=jnp.float32)
    t = t + jnp.dot(x_ref[...], w1r_ref[...], preferred_element_type=jnp.float32)
    t = t + b1_ref[...]
    nrm = jnp.sqrt(jnp.sum(t * t, axis=1, keepdims=True))
    h = jnp.maximum(t / jnp.maximum(nrm, 1e-12), 0.0)
    p_ref[...] = jnp.dot(h, w2_ref[...], preferred_element_type=jnp.float32)


def _tc1(S1p, C1p, x, W1_l, W1_r, b1r, W2cat):
    return pl.pallas_call(
        _tc1_body,
        grid=(N // BN,),
        in_specs=[
            pl.BlockSpec((NC, BN, D), lambda i: (0, i, 0)),
            pl.BlockSpec((NC, BN, 16), lambda i: (0, i, 0)),
            pl.BlockSpec((BN, D), lambda i: (i, 0)),
            pl.BlockSpec((D, H), lambda i: (0, 0)),
            pl.BlockSpec((D, H), lambda i: (0, 0)),
            pl.BlockSpec((1, H), lambda i: (0, 0)),
            pl.BlockSpec((H, 16), lambda i: (0, 0)),
        ],
        out_specs=pl.BlockSpec((BN, 16), lambda i: (i, 0)),
        out_shape=jax.ShapeDtypeStruct((N, 16), jnp.float32),
    )(S1p, C1p, x, W1_l, W1_r, b1r, W2cat)


def _tc2_body(s2_ref, c_ref, p_ref, b2_ref, o_ref):
    S2 = s2_ref[0] + s2_ref[1]
    cnt = c_ref[0, :, 0:1] + c_ref[1, :, 0:1]
    agg = S2[:, 0:2] / jnp.maximum(cnt, 1.0)
    v = agg + p_ref[:, 2:4] + b2_ref[...]
    nrm = jnp.sqrt(jnp.sum(v * v, axis=1, keepdims=True))
    v = v / jnp.maximum(nrm, 1e-12)
    m = jnp.max(v, axis=1, keepdims=True)
    e = jnp.exp(v - m)
    o_ref[...] = (v - m) - jnp.log(jnp.sum(e, axis=1, keepdims=True))


def _tc2(S2p, C1p, p2s, b2r):
    return pl.pallas_call(
        _tc2_body,
        grid=(N // BN,),
        in_specs=[
            pl.BlockSpec((NC, BN, 16), lambda i: (0, i, 0)),
            pl.BlockSpec((NC, BN, 16), lambda i: (0, i, 0)),
            pl.BlockSpec((BN, 16), lambda i: (i, 0)),
            pl.BlockSpec((1, OUT), lambda i: (0, 0)),
        ],
        out_specs=pl.BlockSpec((BN, OUT), lambda i: (i, 0)),
        out_shape=jax.ShapeDtypeStruct((N, OUT), jnp.float32),
    )(S2p, C1p, p2s, b2r)


def kernel(x, edge_index, W1_l, W1_r, b1, W2_l, W2_r, b2):
    src = edge_index[0].reshape(ER, CH)
    dst = edge_index[1].reshape(ER, CH)
    S1p = _segsum128(x, src, dst)
    C1p = _counts(dst)
    W2cat = (jnp.zeros((H, 16), jnp.float32)
             .at[:, 0:2].set(W2_l).at[:, 2:4].set(W2_r))
    p2s = _tc1(S1p, C1p, x, W1_l, W1_r, b1.reshape(1, H), W2cat)
    S2p = _segsum16(p2s, src, dst)
    if isinstance(S2p, (list, tuple)):
        S2p = S2p[0]
    out = _tc2(S2p, C1p, p2s, b2.reshape(1, OUT))
    return out


# speedup vs baseline: 15.3609x; 1.0150x over previous
import jax
import jax.numpy as jnp
from jax import lax
from jax.experimental import pallas as pl
from jax.experimental.pallas import tpu as pltpu
from jax.experimental.pallas import tpu_sc as plsc

N = 10000
E = 320000
D = 128
H = 128
OUT = 2

NC = 2
NS = 16
NW = NC * NS
NP = 10240
RPT = NP // NS
ZCH = 16
OCH = 128

CH = 128
EPW = E // NW
NCH = EPW // CH
SLAB = 26
NSLAB = NCH // SLAB
TAIL = EPW - NCH * CH

CH2 = 512
NCH2 = E // CH2
BASE_HI = 17 * 20


def _zero_acc(stage, acc, s, width, sem):
    zv = jnp.zeros((16,), jnp.float32)

    def zrow(i, _):
        for g in range(width // 16):
            stage[i, pl.ds(g * 16, 16)] = zv
        return 0
    lax.fori_loop(0, ZCH, zrow, 0)
    hs = []
    for r in range(RPT // ZCH):
        hs.append(pltpu.async_copy(
            stage, acc.at[pl.ds(s * RPT + r * ZCH, ZCH)], sem))
    for h in hs:
        h.wait()


def _copy_out(acc, out, c, s, sem):
    hs = []
    for r in range(RPT // OCH):
        off = s * RPT + r * OCH
        hs.append(pltpu.async_copy(acc.at[pl.ds(off, OCH)],
                                   out.at[c, pl.ds(off, OCH)], sem))
    for h in hs:
        h.wait()


def _make_segsum128():
    mesh = plsc.VectorSubcoreMesh(core_axis_name="c", subcore_axis_name="s")
    out_type = jax.ShapeDtypeStruct((NC, NP, D), jnp.float32)
    scratch = [
        pltpu.VMEM((SLAB * CH,), jnp.int32),
        pltpu.VMEM((SLAB * CH,), jnp.int32),
        pltpu.VMEM((CH, D), jnp.float32),
        pltpu.VMEM((CH, D), jnp.float32),
        pltpu.VMEM((ZCH, D), jnp.float32),
        pltpu.VMEM_SHARED((NP, D), jnp.float32),
        pltpu.SemaphoreType.DMA,
        pltpu.SemaphoreType.DMA,
        pltpu.SemaphoreType.DMA,
        pltpu.SemaphoreType.DMA,
        pltpu.SemaphoreType.DMA,
    ]

    def body(x_hbm, src_hbm, dst_hbm, s_out, srcs, dsts, rows0, rows1,
             stage, acc, sg0, sg1, ss0, ss1, sz):
        c = lax.axis_index("c")
        s = lax.axis_index("s")
        w = c * NS + s
        rows = (rows0, rows1)
        sg = (sg0, sg1)
        ss = (ss0, ss1)

        _zero_acc(stage, acc, s, D, sz)
        plsc.subcore_barrier()

        ebase = w * EPW

        def slab(t, _):
            fbase = pl.multiple_of(ebase + t * (SLAB * CH), 8)
            pltpu.sync_copy(src_hbm.at[pl.ds(fbase, SLAB * CH)], srcs)
            pltpu.sync_copy(dst_hbm.at[pl.ds(fbase, SLAB * CH)], dsts)
            gd = [None, None]
            sd = [None, None]
            gd[0] = pltpu.async_copy(
                x_hbm.at[srcs.at[pl.ds(0, CH)]], rows0, sg0)
            for b in range(SLAB):
                cur = b % 2
                nxt = 1 - cur
                if b + 1 < SLAB:
                    if sd[nxt] is not None:
                        sd[nxt].wait()
                    gd[nxt] = pltpu.async_copy(
                        x_hbm.at[srcs.at[pl.ds((b + 1) * CH, CH)]],
                        rows[nxt], sg[nxt])
                gd[cur].wait()
                sd[cur] = pltpu.async_copy(
                    rows[cur], acc.at[dsts.at[pl.ds(b * CH, CH)]],
                    ss[cur], add=True)
            for slot in range(2):
                if sd[slot] is not None:
                    sd[slot].wait()
            return 0
        lax.fori_loop(0, NSLAB, slab, 0)

        tbase = pl.multiple_of(ebase + NCH * CH, 8)
        pltpu.sync_copy(src_hbm.at[pl.ds(tbase, TAIL)],
                        srcs.at[pl.ds(0, TAIL)])
        pltpu.sync_copy(dst_hbm.at[pl.ds(tbase, TAIL)],
                        dsts.at[pl.ds(0, TAIL)])
        pltpu.async_copy(x_hbm.at[srcs.at[pl.ds(0, TAIL)]],
                         rows0.at[pl.ds(0, TAIL)], sg0).wait()
        pltpu.sync_copy(rows0.at[pl.ds(0, TAIL)],
                        acc.at[dsts.at[pl.ds(0, TAIL)]], add=True)

        plsc.subcore_barrier()
        _copy_out(acc, s_out, c, s, sz)

    params = pltpu.CompilerParams(use_tc_tiling_on_sc=False)
    return pl.kernel(body, out_type=out_type, mesh=mesh,
                     scratch_types=scratch, compiler_params=params)


def _w2_chunk_base(w):
    return jnp.where(w < 17, 20 * w, BASE_HI + 19 * (w - 17))


def _make_segsum16():
    NSLOT = 4
    mesh = plsc.VectorSubcoreMesh(core_axis_name="c", subcore_axis_name="s")
    out_type = jax.ShapeDtypeStruct((NC, NP, 16), jnp.float32)
    scratch = [
        pltpu.VMEM((20 * CH2,), jnp.int32),
        pltpu.VMEM((20 * CH2,), jnp.int32),
        [pltpu.VMEM((CH2, 16), jnp.float32) for _ in range(NSLOT)],
        pltpu.VMEM((ZCH, 16), jnp.float32),
        pltpu.VMEM_SHARED((N, 16), jnp.float32),
        pltpu.VMEM_SHARED((NP, 16), jnp.float32),
        [pltpu.SemaphoreType.DMA for _ in range(NSLOT)],
        [pltpu.SemaphoreType.DMA for _ in range(NSLOT)],
        pltpu.SemaphoreType.DMA,
    ]
    TPT = N // NS

    def body(x_hbm, src_hbm, dst_hbm, s_out, srcs, dsts, rows, stage,
             tbl, acc, sg, ss, sz):
        c = lax.axis_index("c")
        s = lax.axis_index("s")
        w = c * NS + s

        _zero_acc(stage, acc, s, 16, sz)
        pltpu.async_copy(x_hbm.at[pl.ds(s * TPT, TPT)],
                         tbl.at[pl.ds(s * TPT, TPT)], sz).wait()
        ebase = pl.multiple_of(_w2_chunk_base(w) * CH2, 8)
        pltpu.sync_copy(src_hbm.at[pl.ds(ebase, 19 * CH2)],
                        srcs.at[pl.ds(0, 19 * CH2)])
        pltpu.sync_copy(dst_hbm.at[pl.ds(ebase, 19 * CH2)],
                        dsts.at[pl.ds(0, 19 * CH2)])

        @pl.when(w < 17)
        def _():
            xb = pl.multiple_of(ebase + 19 * CH2, 8)
            pltpu.sync_copy(src_hbm.at[pl.ds(xb, CH2)],
                            srcs.at[pl.ds(19 * CH2, CH2)])
            pltpu.sync_copy(dst_hbm.at[pl.ds(xb, CH2)],
                            dsts.at[pl.ds(19 * CH2, CH2)])
        plsc.subcore_barrier()

        LA = NSLOT - 1
        gd = [None] * NSLOT
        sd = [None] * NSLOT

        def gop(b, slot):
            return pltpu.async_copy(
                tbl.at[srcs.at[pl.ds(b * CH2, CH2)]], rows[slot], sg[slot])

        for k in range(LA):
            gd[k] = gop(k, k)
        for b in range(19):
            cur = b % NSLOT
            nb = b + LA
            if nb < 19:
                slot = nb % NSLOT
                if sd[slot] is not None:
                    sd[slot].wait()
                gd[slot] = gop(nb, slot)
            gd[cur].wait()
            sd[cur] = pltpu.async_copy(
                rows[cur], acc.at[dsts.at[pl.ds(b * CH2, CH2)]],
                ss[cur], add=True)
        for slot in range(NSLOT):
            if sd[slot] is not None:
                sd[slot].wait()

        @pl.when(w < 17)
        def _():
            gop(19, 0).wait()
            pltpu.sync_copy(rows[0], acc.at[dsts.at[pl.ds(19 * CH2, CH2)]],
                            add=True)

        plsc.subcore_barrier()
        _copy_out(acc, s_out, c, s, sz)

    params = pltpu.CompilerParams(use_tc_tiling_on_sc=False)
    return pl.kernel(body, out_type=out_type, mesh=mesh,
                     scratch_types=scratch, compiler_params=params)


def _make_counts():
    mesh = plsc.VectorSubcoreMesh(core_axis_name="c", subcore_axis_name="s")
    out_type = jax.ShapeDtypeStruct((NC, NP, 16), jnp.float32)
    scratch = [
        pltpu.VMEM((20 * CH2,), jnp.int32),
        pltpu.VMEM((CH2, 16), jnp.float32),
        pltpu.VMEM((ZCH, 16), jnp.float32),
        pltpu.VMEM_SHARED((NP, 16), jnp.float32),
        pltpu.SemaphoreType.DMA,
        pltpu.SemaphoreType.DMA,
    ]

    def body(dst_hbm, c_out, dsts, ones, stage, cacc, sc, sz):
        c = lax.axis_index("c")
        s = lax.axis_index("s")
        w = c * NS + s
        ov = jnp.ones((16,), jnp.float32)

        def orow(i, _):
            ones[i, :] = ov
            return 0
        lax.fori_loop(0, CH2, orow, 0)
        _zero_acc(stage, cacc, s, 16, sz)

        ebase = pl.multiple_of(_w2_chunk_base(w) * CH2, 8)
        pltpu.sync_copy(dst_hbm.at[pl.ds(ebase, 19 * CH2)],
                        dsts.at[pl.ds(0, 19 * CH2)])

        @pl.when(w < 17)
        def _():
            xb = pl.multiple_of(ebase + 19 * CH2, 8)
            pltpu.sync_copy(dst_hbm.at[pl.ds(xb, CH2)],
                            dsts.at[pl.ds(19 * CH2, CH2)])
        plsc.subcore_barrier()

        sd = []
        for b in range(19):
            sd.append(pltpu.async_copy(
                ones, cacc.at[dsts.at[pl.ds(b * CH2, CH2)]], sc, add=True))
        for h in sd:
            h.wait()

        @pl.when(w < 17)
        def _():
            pltpu.sync_copy(ones, cacc.at[dsts.at[pl.ds(19 * CH2, CH2)]],
                            add=True)

        plsc.subcore_barrier()
        _copy_out(cacc, c_out, c, s, sz)

    params = pltpu.CompilerParams(use_tc_tiling_on_sc=False)
    return pl.kernel(body, out_type=out_type, mesh=mesh,
                     scratch_types=scratch, compiler_params=params)


_segsum128 = _make_segsum128()
_segsum16 = _make_segsum16()
_counts = _make_counts()

BN = 2000


def _tc1_body(s_ref, c_ref, x_ref, w1l_ref, w1r_ref, b1_ref, w2_ref, p_ref):
    S = s_ref[0] + s_ref[1]
    cnt = c_ref[0, :, 0:1] + c_ref[1, :, 0:1]
    agg = S / jnp.maximum(cnt, 1.0)
    t = jnp.dot(agg, w1l_ref[...], preferred_element_type=jnp.float32)
    t = t + jnp.dot(x_ref[...], w1r_ref[...], preferred_element_type=jnp.float32)
    t = t + b1_ref[...]
    nrm = jnp.sqrt(jnp.sum(t * t, axis=1, keepdims=True))
    h = jnp.maximum(t / jnp.maximum(nrm, 1e-12), 0.0)
    p_ref[...] = jnp.dot(h, w2_ref[...], preferred_element_type=jnp.float32)


def _tc1(S1p, C1p, x, W1_l, W1_r, b1r, W2cat):
    return pl.pallas_call(
        _tc1_body,
        grid=(N // BN,),
        in_specs=[
            pl.BlockSpec((NC, BN, D), lambda i: (0, i, 0)),
            pl.BlockSpec((NC, BN, 16), lambda i: (0, i, 0)),
            pl.BlockSpec((BN, D), lambda i: (i, 0)),
            pl.BlockSpec((D, H), lambda i: (0, 0)),
            pl.BlockSpec((D, H), lambda i: (0, 0)),
            pl.BlockSpec((1, H), lambda i: (0, 0)),
            pl.BlockSpec((H, 16), lambda i: (0, 0)),
        ],
        out_specs=pl.BlockSpec((BN, 16), lambda i: (i, 0)),
        out_shape=jax.ShapeDtypeStruct((N, 16), jnp.float32),
    )(S1p, C1p, x, W1_l, W1_r, b1r, W2cat)


def _tc2_body(s2_ref, c_ref, p_ref, b2_ref, o_ref):
    S2 = s2_ref[0] + s2_ref[1]
    cnt = c_ref[0, :, 0:1] + c_ref[1, :, 0:1]
    agg = S2[:, 0:2] / jnp.maximum(cnt, 1.0)
    v = agg + p_ref[:, 2:4] + b2_ref[...]
    nrm = jnp.sqrt(jnp.sum(v * v, axis=1, keepdims=True))
    v = v / jnp.maximum(nrm, 1e-12)
    m = jnp.max(v, axis=1, keepdims=True)
    e = jnp.exp(v - m)
    o_ref[...] = (v - m) - jnp.log(jnp.sum(e, axis=1, keepdims=True))


def _tc2(S2p, C1p, p2s, b2r):
    return pl.pallas_call(
        _tc2_body,
        grid=(N // BN,),
        in_specs=[
            pl.BlockSpec((NC, BN, 16), lambda i: (0, i, 0)),
            pl.BlockSpec((NC, BN, 16), lambda i: (0, i, 0)),
            pl.BlockSpec((BN, 16), lambda i: (i, 0)),
            pl.BlockSpec((1, OUT), lambda i: (0, 0)),
        ],
        out_specs=pl.BlockSpec((BN, OUT), lambda i: (i, 0)),
        out_shape=jax.ShapeDtypeStruct((N, OUT), jnp.float32),
    )(S2p, C1p, p2s, b2r)


def kernel(x, edge_index, W1_l, W1_r, b1, W2_l, W2_r, b2):
    src = edge_index[0]
    dst = edge_index[1]
    S1p = _segsum128(x, src, dst)
    C1p = _counts(dst)
    W2cat = (jnp.zeros((H, 16), jnp.float32)
             .at[:, 0:2].set(W2_l).at[:, 2:4].set(W2_r))
    p2s = _tc1(S1p, C1p, x, W1_l, W1_r, b1.reshape(1, H), W2cat)
    S2p = _segsum16(p2s, src, dst)
    out = _tc2(S2p, C1p, p2s, b2.reshape(1, OUT))
    return out
